# Initial kernel scaffold; baseline (speedup 1.0000x reference)
import jax
import jax.numpy as jnp
from jax.experimental import pallas as pl


def kernel(x, edge_index, W0, b0, W1, b1, W2, b2, g0, be0, g1, be1, g2, be2, Wc1, bc1, Wc2, bc2, Wr1, br1, Wr2, br2):
    raise NotImplementedError("write your pallas kernel here")


# speedup vs baseline: 1.0839x; 1.0839x over previous
import functools

import jax
import jax.numpy as jnp
from jax.experimental import pallas as pl

N = 10000
EPS = 1e-5


def _mm_kernel(x_ref, w_ref, o_ref):
    o_ref[...] = jnp.dot(x_ref[...], w_ref[...],
                         preferred_element_type=jnp.float32)


def _pl_matmul(x, W):
    M, K = x.shape
    blk = 2000
    return pl.pallas_call(
        _mm_kernel,
        grid=(M // blk,),
        in_specs=[pl.BlockSpec((blk, K), lambda i: (i, 0)),
                  pl.BlockSpec((K, W.shape[1]), lambda i: (0, 0))],
        out_specs=pl.BlockSpec((blk, W.shape[1]), lambda i: (i, 0)),
        out_shape=jax.ShapeDtypeStruct((M, W.shape[1]), jnp.float32),
    )(x, W)


def _gcn_conv(x, src, dst, W, b):
    h = _pl_matmul(x, W)
    deg = jnp.zeros((N,), jnp.float32).at[dst].add(1.0)
    dinv = jnp.where(deg > 0, 1.0 / jnp.sqrt(deg), 0.0)
    norm = dinv[src] * dinv[dst]
    msg = h[src] * norm[:, None]
    out = jnp.zeros((N, W.shape[1]), jnp.float32).at[dst].add(msg)
    return out + b


def _bn(x, g, b):
    m = jnp.mean(x, axis=0)
    v = jnp.var(x, axis=0)
    return (x - m) / jnp.sqrt(v + EPS) * g + b


def kernel(x, edge_index, W0, b0, W1, b1, W2, b2, g0, be0, g1, be1, g2, be2,
           Wc1, bc1, Wc2, bc2, Wr1, br1, Wr2, br2):
    loop = jnp.arange(N, dtype=edge_index.dtype)
    src = jnp.concatenate([edge_index[0], loop])
    dst = jnp.concatenate([edge_index[1], loop])
    h = _gcn_conv(x, src, dst, W0, b0)
    h = jax.nn.relu(_bn(h, g0, be0))
    h = _gcn_conv(h, src, dst, W1, b1)
    h = jax.nn.relu(_bn(h, g1, be1))
    h = _gcn_conv(h, src, dst, W2, b2)
    h = _bn(h, g2, be2)
    ge = jnp.mean(h, axis=0, keepdims=True)
    logits = jax.nn.relu(ge @ Wc1 + bc1) @ Wc2 + bc2
    reg = jax.nn.sigmoid(jax.nn.relu(ge @ Wr1 + br1) @ Wr2 + br2)
    return (logits, reg)
